# baseline (speedup 1.0000x reference)
import functools

import jax
import jax.numpy as jnp
from jax import lax
from jax.experimental import pallas as pl
from jax.experimental.pallas import tpu as pltpu
from jax.experimental.pallas import tpu_sc as plsc

_NC = 2
_NS = 16
_NW = _NC * _NS
_NL = 16

_SCHUNK = 32


def _sc_body(x_hbm, pos_hbm, out_hbm, pos_v, x_v):
    B, S, D = x_hbm.shape
    wid = lax.axis_index("s") * _NC + lax.axis_index("c")
    chunks_per_w = (S // _SCHUNK) // _NW

    def s_body(i, carry):
        s0 = (wid * chunks_per_w + i) * _SCHUNK
        pltpu.sync_copy(pos_hbm.at[pl.ds(s0, _SCHUNK)], pos_v)

        def b_body(b, carry):
            pltpu.sync_copy(x_hbm.at[b, pl.ds(s0, _SCHUNK)], x_v)

            def r_body(r, carry):
                for c in range(D // _NL):
                    sl = pl.ds(c * _NL, _NL)
                    plsc.addupdate(x_v.at[r, sl], pos_v[r, sl])
                return carry

            lax.fori_loop(0, _SCHUNK, r_body, 0, unroll=False)
            pltpu.sync_copy(x_v, out_hbm.at[b, pl.ds(s0, _SCHUNK)])
            return carry

        lax.fori_loop(0, B, b_body, 0, unroll=False)
        return carry

    lax.fori_loop(0, chunks_per_w, s_body, 0, unroll=False)


def kernel(x, pos_table):
    B, S, D = x.shape
    mesh = plsc.VectorSubcoreMesh(core_axis_name="c", subcore_axis_name="s")
    k = functools.partial(
        pl.kernel,
        mesh=mesh,
        out_type=jax.ShapeDtypeStruct((B, S, D), x.dtype),
        scratch_types=[
            pltpu.VMEM((_SCHUNK, D), jnp.float32),
            pltpu.VMEM((_SCHUNK, D), jnp.float32),
        ],
    )(_sc_body)
    return k(x, pos_table)


# speedup vs baseline: 1.2336x; 1.2336x over previous
import functools

import jax
import jax.numpy as jnp
from jax import lax
from jax.experimental import pallas as pl
from jax.experimental.pallas import tpu as pltpu
from jax.experimental.pallas import tpu_sc as plsc

_NC = 2
_NS = 16
_NW = _NC * _NS
_NL = 16

_SCHUNK = 32


def _tc_body(x_ref, pos_ref, o_ref):
    o_ref[...] = x_ref[...] + pos_ref[...]


def _tc_part(x, pos_table):
    B, S, D = x.shape
    return pl.pallas_call(
        _tc_body,
        grid=(B,),
        in_specs=[
            pl.BlockSpec((1, S, D), lambda b: (b, 0, 0)),
            pl.BlockSpec((S, D), lambda b: (0, 0)),
        ],
        out_specs=pl.BlockSpec((1, S, D), lambda b: (b, 0, 0)),
        out_shape=jax.ShapeDtypeStruct((B, S, D), x.dtype),
    )(x, pos_table)


def _sc_body(x_hbm, pos_hbm, out_hbm, pos_v, x_v):
    B, S, D = x_hbm.shape
    wid = lax.axis_index("s") * _NC + lax.axis_index("c")
    chunks_per_w = (B * S // _SCHUNK) // _NW

    def s_body(i, carry):
        row0 = (wid * chunks_per_w + i) * _SCHUNK
        b = row0 // S
        s0 = row0 % S
        pltpu.sync_copy(pos_hbm.at[pl.ds(s0, _SCHUNK)], pos_v)
        pltpu.sync_copy(x_hbm.at[b, pl.ds(s0, _SCHUNK)], x_v)

        def r_body(r, carry):
            for c in range(D // _NL):
                sl = pl.ds(c * _NL, _NL)
                plsc.addupdate(x_v.at[r, sl], pos_v[r, sl])
            return carry

        lax.fori_loop(0, _SCHUNK, r_body, 0, unroll=False)
        pltpu.sync_copy(x_v, out_hbm.at[b, pl.ds(s0, _SCHUNK)])
        return carry

    lax.fori_loop(0, chunks_per_w, s_body, 0, unroll=False)


def _sc_part(x, pos_table):
    B, S, D = x.shape
    mesh = plsc.VectorSubcoreMesh(core_axis_name="c", subcore_axis_name="s")
    k = functools.partial(
        pl.kernel,
        mesh=mesh,
        out_type=jax.ShapeDtypeStruct((B, S, D), x.dtype),
        scratch_types=[
            pltpu.VMEM((_SCHUNK, D), jnp.float32),
            pltpu.VMEM((_SCHUNK, D), jnp.float32),
        ],
    )(_sc_body)
    return k(x, pos_table)


def kernel(x, pos_table):
    B, S, D = x.shape
    out_tc = _tc_part(x[: B - 1], pos_table)
    out_sc = _sc_part(x[B - 1 :], pos_table)
    return jnp.concatenate([out_tc, out_sc], axis=0)


# speedup vs baseline: 4.6073x; 3.7349x over previous
import jax
import jax.numpy as jnp
from jax.experimental import pallas as pl


def _add_body(x_ref, pos_ref, o_ref):
    o_ref[...] = x_ref[...] + pos_ref[...]


def kernel(x, pos_table):
    B, S, D = x.shape
    return pl.pallas_call(
        _add_body,
        grid=(B,),
        in_specs=[
            pl.BlockSpec((1, S, D), lambda b: (b, 0, 0)),
            pl.BlockSpec((S, D), lambda b: (0, 0)),
        ],
        out_specs=pl.BlockSpec((1, S, D), lambda b: (b, 0, 0)),
        out_shape=jax.ShapeDtypeStruct((B, S, D), x.dtype),
    )(x, pos_table)


# speedup vs baseline: 4.6116x; 1.0010x over previous
import jax
from jax.experimental import pallas as pl


def _add_body(x_ref, pos_ref, o_ref):
    o_ref[...] = x_ref[...] + pos_ref[...]


def kernel(x, pos_table):
    B, S, D = x.shape
    return pl.pallas_call(
        _add_body,
        grid=(B,),
        in_specs=[
            pl.BlockSpec((1, S, D), lambda b: (b, 0, 0)),
            pl.BlockSpec((S, D), lambda b: (0, 0)),
        ],
        out_specs=pl.BlockSpec((1, S, D), lambda b: (b, 0, 0)),
        out_shape=jax.ShapeDtypeStruct((B, S, D), x.dtype),
    )(x, pos_table)
